# initial kernel scaffold (unmeasured)
import jax
import jax.numpy as jnp
from jax import lax
from jax.experimental import pallas as pl
from jax.experimental.pallas import tpu as pltpu


def kernel(
    u,
):
    def body(*refs):
        pass

    out_shape = jax.ShapeDtypeStruct(..., jnp.float32)
    return pl.pallas_call(body, out_shape=out_shape)(...)


# baseline (device time: 7031 ns/iter reference)
import jax
import jax.numpy as jnp
from jax import lax
from jax.experimental import pallas as pl
from jax.experimental.pallas import tpu as pltpu

MESH_SHAPE = (2, 4, 4)


def kernel(u):
    sx, sy, sz = u.shape

    def body(u_ref, out_ref, sbx, sby, sbz, rbx, rby, rbz, send_sems, recv_sems):
        mx = lax.axis_index("x")
        my = lax.axis_index("y")
        mz = lax.axis_index("z")
        pos = (mx, my, mz)
        sbufs = (sbx, sby, sbz)
        rbufs = (rbx, rby, rbz)

        has = []
        nbr = []
        for ax in range(3):
            for sign in range(2):
                step = 1 if sign else -1
                has.append(
                    pos[ax] < MESH_SHAPE[ax] - 1 if sign else pos[ax] > 0
                )
                nbr.append(
                    tuple(pos[a] + (step if a == ax else 0) for a in range(3))
                )

        uu = u_ref[...]
        sbx[0] = uu[0, :, :]
        sbx[1] = uu[sx - 1, :, :]
        sby[0] = uu[:, 0, :]
        sby[1] = uu[:, sy - 1, :]
        sbz[0] = uu[:, :, 0]
        sbz[1] = uu[:, :, sz - 1]

        bar = pltpu.get_barrier_semaphore()
        for d in range(6):
            @pl.when(has[d])
            def _(d=d):
                pl.semaphore_signal(
                    bar, inc=1, device_id=nbr[d],
                    device_id_type=pl.DeviceIdType.MESH,
                )

            @pl.when(jnp.logical_not(has[d]))
            def _():
                pl.semaphore_signal(bar, inc=1)
        pl.semaphore_wait(bar, 6)

        def send_desc(d):
            ax, sign = d // 2, d % 2
            return pltpu.make_async_remote_copy(
                src_ref=sbufs[ax].at[sign],
                dst_ref=rbufs[ax].at[1 - sign],
                send_sem=send_sems.at[d],
                recv_sem=recv_sems.at[d ^ 1],
                device_id=nbr[d],
                device_id_type=pl.DeviceIdType.MESH,
            )

        def recv_desc(d):
            ax, sign = d // 2, d % 2
            return pltpu.make_async_remote_copy(
                src_ref=sbufs[ax].at[sign],
                dst_ref=rbufs[ax].at[sign],
                send_sem=send_sems.at[d],
                recv_sem=recv_sems.at[d],
                device_id=nbr[d],
                device_id_type=pl.DeviceIdType.MESH,
            )

        for d in range(6):
            @pl.when(has[d])
            def _(d=d):
                send_desc(d).start()

        for d in range(6):
            @pl.when(has[d])
            def _(d=d):
                send_desc(d).wait_send()
                recv_desc(d).wait_recv()

        dn_x = jnp.concatenate([rbx[0][None, :, :], uu[:-1]], axis=0)
        up_x = jnp.concatenate([uu[1:], rbx[1][None, :, :]], axis=0)
        dn_y = jnp.concatenate([rby[0][:, None, :], uu[:, :-1]], axis=1)
        up_y = jnp.concatenate([uu[:, 1:], rby[1][:, None, :]], axis=1)
        dn_z = jnp.concatenate([rbz[0][:, :, None], uu[:, :, :-1]], axis=2)
        up_z = jnp.concatenate([uu[:, :, 1:], rbz[1][:, :, None]], axis=2)
        v = dn_x + up_x + dn_y + up_y + dn_z + up_z - 6.0 * uu

        ix = lax.broadcasted_iota(jnp.int32, (sx, sy, sz), 0)
        iy = lax.broadcasted_iota(jnp.int32, (sx, sy, sz), 1)
        iz = lax.broadcasted_iota(jnp.int32, (sx, sy, sz), 2)
        bmask = (
            ((mx == 0) & (ix == 0))
            | ((mx == MESH_SHAPE[0] - 1) & (ix == sx - 1))
            | ((my == 0) & (iy == 0))
            | ((my == MESH_SHAPE[1] - 1) & (iy == sy - 1))
            | ((mz == 0) & (iz == 0))
            | ((mz == MESH_SHAPE[2] - 1) & (iz == sz - 1))
        )
        out_ref[...] = jnp.where(bmask, 0.0, v)

    return pl.pallas_call(
        body,
        out_shape=jax.ShapeDtypeStruct((sx, sy, sz), u.dtype),
        in_specs=[pl.BlockSpec(memory_space=pltpu.VMEM)],
        out_specs=pl.BlockSpec(memory_space=pltpu.VMEM),
        scratch_shapes=[
            pltpu.VMEM((2, sy, sz), u.dtype),
            pltpu.VMEM((2, sx, sz), u.dtype),
            pltpu.VMEM((2, sx, sy), u.dtype),
            pltpu.VMEM((2, sy, sz), u.dtype),
            pltpu.VMEM((2, sx, sz), u.dtype),
            pltpu.VMEM((2, sx, sy), u.dtype),
            pltpu.SemaphoreType.DMA((6,)),
            pltpu.SemaphoreType.DMA((6,)),
        ],
        compiler_params=pltpu.CompilerParams(collective_id=0),
    )(u)


# device time: 6877 ns/iter; 1.0224x vs baseline; 1.0224x over previous
import jax
import jax.numpy as jnp
from jax import lax
from jax.experimental import pallas as pl
from jax.experimental.pallas import tpu as pltpu

MESH_SHAPE = (2, 4, 4)


def kernel(u):
    sx, sy, sz = u.shape

    def body(u_ref, out_ref, sbx, sby, sbz, rbx, rby, rbz, send_sems, recv_sems):
        mx = lax.axis_index("x")
        my = lax.axis_index("y")
        mz = lax.axis_index("z")
        pos = (mx, my, mz)
        sbufs = (sbx, sby, sbz)
        rbufs = (rbx, rby, rbz)

        has = []
        nbr = []
        for ax in range(3):
            for sign in range(2):
                step = 1 if sign else -1
                has.append(
                    pos[ax] < MESH_SHAPE[ax] - 1 if sign else pos[ax] > 0
                )
                nbr.append(
                    tuple(pos[a] + (step if a == ax else 0) for a in range(3))
                )

        bar = pltpu.get_barrier_semaphore()
        for d in range(6):
            @pl.when(has[d])
            def _(d=d):
                pl.semaphore_signal(
                    bar, inc=1, device_id=nbr[d],
                    device_id_type=pl.DeviceIdType.MESH,
                )

            @pl.when(jnp.logical_not(has[d]))
            def _():
                pl.semaphore_signal(bar, inc=1)

        uu = u_ref[...]
        sbx[0] = uu[0, :, :]
        sbx[1] = uu[sx - 1, :, :]
        sby[0] = uu[:, 0, :]
        sby[1] = uu[:, sy - 1, :]
        sbz[0] = uu[:, :, 0]
        sbz[1] = uu[:, :, sz - 1]

        pl.semaphore_wait(bar, 6)

        def send_desc(d):
            ax, sign = d // 2, d % 2
            return pltpu.make_async_remote_copy(
                src_ref=sbufs[ax].at[sign],
                dst_ref=rbufs[ax].at[1 - sign],
                send_sem=send_sems.at[d],
                recv_sem=recv_sems.at[d ^ 1],
                device_id=nbr[d],
                device_id_type=pl.DeviceIdType.MESH,
            )

        def recv_desc(d):
            ax, sign = d // 2, d % 2
            return pltpu.make_async_remote_copy(
                src_ref=sbufs[ax].at[sign],
                dst_ref=rbufs[ax].at[sign],
                send_sem=send_sems.at[d],
                recv_sem=recv_sems.at[d],
                device_id=nbr[d],
                device_id_type=pl.DeviceIdType.MESH,
            )

        for d in range(6):
            @pl.when(has[d])
            def _(d=d):
                send_desc(d).start()

        zface_yz = jnp.zeros((1, sy, sz), uu.dtype)
        zface_xz = jnp.zeros((sx, 1, sz), uu.dtype)
        zface_xy = jnp.zeros((sx, sy, 1), uu.dtype)
        dn_x = jnp.concatenate([zface_yz, uu[:-1]], axis=0)
        up_x = jnp.concatenate([uu[1:], zface_yz], axis=0)
        dn_y = jnp.concatenate([zface_xz, uu[:, :-1]], axis=1)
        up_y = jnp.concatenate([uu[:, 1:], zface_xz], axis=1)
        dn_z = jnp.concatenate([zface_xy, uu[:, :, :-1]], axis=2)
        up_z = jnp.concatenate([uu[:, :, 1:], zface_xy], axis=2)
        v = dn_x + up_x + dn_y + up_y + dn_z + up_z - 6.0 * uu

        ix = lax.broadcasted_iota(jnp.int32, (sx, sy, sz), 0)
        iy = lax.broadcasted_iota(jnp.int32, (sx, sy, sz), 1)
        iz = lax.broadcasted_iota(jnp.int32, (sx, sy, sz), 2)
        bmask = (
            ((mx == 0) & (ix == 0))
            | ((mx == MESH_SHAPE[0] - 1) & (ix == sx - 1))
            | ((my == 0) & (iy == 0))
            | ((my == MESH_SHAPE[1] - 1) & (iy == sy - 1))
            | ((mz == 0) & (iz == 0))
            | ((mz == MESH_SHAPE[2] - 1) & (iz == sz - 1))
        )

        for d in range(6):
            @pl.when(has[d])
            def _(d=d):
                recv_desc(d).wait_recv()

        v = v + jnp.where(ix == 0, rbx[0][None, :, :], 0.0)
        v = v + jnp.where(ix == sx - 1, rbx[1][None, :, :], 0.0)
        v = v + jnp.where(iy == 0, rby[0][:, None, :], 0.0)
        v = v + jnp.where(iy == sy - 1, rby[1][:, None, :], 0.0)
        v = v + jnp.where(iz == 0, rbz[0][:, :, None], 0.0)
        v = v + jnp.where(iz == sz - 1, rbz[1][:, :, None], 0.0)

        out_ref[...] = jnp.where(bmask, 0.0, v)

        for d in range(6):
            @pl.when(has[d])
            def _(d=d):
                send_desc(d).wait_send()

    return pl.pallas_call(
        body,
        out_shape=jax.ShapeDtypeStruct((sx, sy, sz), u.dtype),
        in_specs=[pl.BlockSpec(memory_space=pltpu.VMEM)],
        out_specs=pl.BlockSpec(memory_space=pltpu.VMEM),
        scratch_shapes=[
            pltpu.VMEM((2, sy, sz), u.dtype),
            pltpu.VMEM((2, sx, sz), u.dtype),
            pltpu.VMEM((2, sx, sy), u.dtype),
            pltpu.VMEM((2, sy, sz), u.dtype),
            pltpu.VMEM((2, sx, sz), u.dtype),
            pltpu.VMEM((2, sx, sy), u.dtype),
            pltpu.SemaphoreType.DMA((6,)),
            pltpu.SemaphoreType.DMA((6,)),
        ],
        compiler_params=pltpu.CompilerParams(collective_id=0),
    )(u)


# device time: 6807 ns/iter; 1.0329x vs baseline; 1.0103x over previous
import jax
import jax.numpy as jnp
from jax import lax
from jax.experimental import pallas as pl
from jax.experimental.pallas import tpu as pltpu

MESH_SHAPE = (2, 4, 4)


def kernel(u):
    sx, sy, sz = u.shape

    def body(
        u_ref, out_ref, sbz, rbx, rby, rbz, send_sems, recv_sems,
    ):
        mx = lax.axis_index("x")
        my = lax.axis_index("y")
        mz = lax.axis_index("z")
        pos = (mx, my, mz)
        rbufs = (rbx, rby, rbz)

        has = []
        nbr = []
        for ax in range(3):
            for sign in range(2):
                step = 1 if sign else -1
                has.append(
                    pos[ax] < MESH_SHAPE[ax] - 1 if sign else pos[ax] > 0
                )
                nbr.append(
                    tuple(pos[a] + (step if a == ax else 0) for a in range(3))
                )

        bar = pltpu.get_barrier_semaphore()
        for d in range(6):
            @pl.when(has[d])
            def _(d=d):
                pl.semaphore_signal(
                    bar, inc=1, device_id=nbr[d],
                    device_id_type=pl.DeviceIdType.MESH,
                )

            @pl.when(jnp.logical_not(has[d]))
            def _():
                pl.semaphore_signal(bar, inc=1)

        uu = u_ref[...]
        sbz[0] = uu[:, :, 0]
        sbz[1] = uu[:, :, sz - 1]

        src_refs = [
            u_ref.at[0],
            u_ref.at[sx - 1],
            u_ref.at[:, 0],
            u_ref.at[:, sy - 1],
            sbz.at[0],
            sbz.at[1],
        ]

        def send_desc(d):
            ax, sign = d // 2, d % 2
            return pltpu.make_async_remote_copy(
                src_ref=src_refs[d],
                dst_ref=rbufs[ax].at[1 - sign],
                send_sem=send_sems.at[d],
                recv_sem=recv_sems.at[d ^ 1],
                device_id=nbr[d],
                device_id_type=pl.DeviceIdType.MESH,
            )

        def recv_desc(d):
            ax, sign = d // 2, d % 2
            return pltpu.make_async_remote_copy(
                src_ref=src_refs[d],
                dst_ref=rbufs[ax].at[sign],
                send_sem=send_sems.at[d],
                recv_sem=recv_sems.at[d],
                device_id=nbr[d],
                device_id_type=pl.DeviceIdType.MESH,
            )

        pl.semaphore_wait(bar, 6)

        for d in range(6):
            @pl.when(has[d])
            def _(d=d):
                send_desc(d).start()

        zface_yz = jnp.zeros((1, sy, sz), uu.dtype)
        zface_xz = jnp.zeros((sx, 1, sz), uu.dtype)
        zface_xy = jnp.zeros((sx, sy, 1), uu.dtype)
        dn_x = jnp.concatenate([zface_yz, uu[:-1]], axis=0)
        up_x = jnp.concatenate([uu[1:], zface_yz], axis=0)
        dn_y = jnp.concatenate([zface_xz, uu[:, :-1]], axis=1)
        up_y = jnp.concatenate([uu[:, 1:], zface_xz], axis=1)
        dn_z = jnp.concatenate([zface_xy, uu[:, :, :-1]], axis=2)
        up_z = jnp.concatenate([uu[:, :, 1:], zface_xy], axis=2)
        v = dn_x + up_x + dn_y + up_y + dn_z + up_z - 6.0 * uu

        ix = lax.broadcasted_iota(jnp.int32, (sx, sy, sz), 0)
        iy = lax.broadcasted_iota(jnp.int32, (sx, sy, sz), 1)
        iz = lax.broadcasted_iota(jnp.int32, (sx, sy, sz), 2)
        bmask = (
            ((mx == 0) & (ix == 0))
            | ((mx == MESH_SHAPE[0] - 1) & (ix == sx - 1))
            | ((my == 0) & (iy == 0))
            | ((my == MESH_SHAPE[1] - 1) & (iy == sy - 1))
            | ((mz == 0) & (iz == 0))
            | ((mz == MESH_SHAPE[2] - 1) & (iz == sz - 1))
        )

        for d in range(6):
            @pl.when(has[d])
            def _(d=d):
                recv_desc(d).wait_recv()

        v = v + jnp.where(ix == 0, rbx[0][None, :, :], 0.0)
        v = v + jnp.where(ix == sx - 1, rbx[1][None, :, :], 0.0)
        v = v + jnp.where(iy == 0, rby[0][:, None, :], 0.0)
        v = v + jnp.where(iy == sy - 1, rby[1][:, None, :], 0.0)
        v = v + jnp.where(iz == 0, rbz[0][:, :, None], 0.0)
        v = v + jnp.where(iz == sz - 1, rbz[1][:, :, None], 0.0)

        out_ref[...] = jnp.where(bmask, 0.0, v)

        for d in range(6):
            @pl.when(has[d])
            def _(d=d):
                send_desc(d).wait_send()

    return pl.pallas_call(
        body,
        out_shape=jax.ShapeDtypeStruct((sx, sy, sz), u.dtype),
        in_specs=[pl.BlockSpec(memory_space=pltpu.VMEM)],
        out_specs=pl.BlockSpec(memory_space=pltpu.VMEM),
        scratch_shapes=[
            pltpu.VMEM((2, sx, sy), u.dtype),
            pltpu.VMEM((2, sy, sz), u.dtype),
            pltpu.VMEM((2, sx, sz), u.dtype),
            pltpu.VMEM((2, sx, sy), u.dtype),
            pltpu.SemaphoreType.DMA((6,)),
            pltpu.SemaphoreType.DMA((6,)),
        ],
        compiler_params=pltpu.CompilerParams(collective_id=0),
    )(u)
